# baseline (device time: 94149 ns/iter reference)
import jax
import jax.numpy as jnp
from jax import lax
from jax.experimental import pallas as pl
from jax.experimental.pallas import tpu as pltpu


def kernel(O, Wo):
    B, S, H, D = O.shape
    HD = H * D
    N = Wo.shape[1]
    S_half = S // 2
    Q = S_half // 2

    CHUNK = 64
    n_j = Q // CHUNK
    n_chunks = B * n_j

    def body(o_ref, wo_ref, out_ref, y_sbuf, y_rbuf, x_sbuf, x_rbuf,
             y_send_sems, y_recv_sems, x_send_sems, x_recv_sems):
        my_x = lax.axis_index("x")
        my_y = lax.axis_index("y")
        peer_y = 1 - my_y
        peer_x = 1 - my_x

        barrier = pltpu.get_barrier_semaphore()
        pl.semaphore_signal(
            barrier, inc=1,
            device_id=(my_x, peer_y), device_id_type=pl.DeviceIdType.MESH,
        )
        pl.semaphore_signal(
            barrier, inc=1,
            device_id=(peer_x, my_y), device_id_type=pl.DeviceIdType.MESH,
        )
        pl.semaphore_wait(barrier, 2)

        def partial_chunk(b, row0):
            acc = jnp.dot(
                o_ref[b, pl.ds(row0, CHUNK), 0, :], wo_ref[0],
                preferred_element_type=jnp.float32,
            )
            for h in range(1, H):
                acc += jnp.dot(
                    o_ref[b, pl.ds(row0, CHUNK), h, :], wo_ref[h],
                    preferred_element_type=jnp.float32,
                )
            return acc

        y_rdmas = []
        for c in range(n_chunks):
            b, j = divmod(c, n_j)
            row0 = peer_y * S_half + my_x * Q + j * CHUNK
            y_sbuf[b, pl.ds(j * CHUNK, CHUNK), :] = partial_chunk(b, row0)
            r = pltpu.make_async_remote_copy(
                src_ref=y_sbuf.at[b, pl.ds(j * CHUNK, CHUNK), :],
                dst_ref=y_rbuf.at[b, pl.ds(j * CHUNK, CHUNK), :],
                send_sem=y_send_sems.at[c],
                recv_sem=y_recv_sems.at[c],
                device_id=(my_x, peer_y),
                device_id_type=pl.DeviceIdType.MESH,
            )
            r.start()
            y_rdmas.append(r)

        for c in range(n_chunks):
            b, j = divmod(c, n_j)
            row0 = my_y * S_half + my_x * Q + j * CHUNK
            out_ref[b, pl.ds(my_x * Q + j * CHUNK, CHUNK), :] = partial_chunk(
                b, row0
            )

        x_rdmas = []
        for c in range(n_chunks):
            b, j = divmod(c, n_j)
            out_row0 = my_x * Q + j * CHUNK
            y_rdmas[c].wait_recv()
            red = (
                out_ref[b, pl.ds(out_row0, CHUNK), :]
                + y_rbuf[b, pl.ds(j * CHUNK, CHUNK), :]
            )
            out_ref[b, pl.ds(out_row0, CHUNK), :] = red
            x_sbuf[b, pl.ds(j * CHUNK, CHUNK), :] = red
            r = pltpu.make_async_remote_copy(
                src_ref=x_sbuf.at[b, pl.ds(j * CHUNK, CHUNK), :],
                dst_ref=x_rbuf.at[b, pl.ds(j * CHUNK, CHUNK), :],
                send_sem=x_send_sems.at[c],
                recv_sem=x_recv_sems.at[c],
                device_id=(peer_x, my_y),
                device_id_type=pl.DeviceIdType.MESH,
            )
            r.start()
            x_rdmas.append(r)

        for c in range(n_chunks):
            b, j = divmod(c, n_j)
            x_rdmas[c].wait_recv()
            out_ref[b, pl.ds(peer_x * Q + j * CHUNK, CHUNK), :] = x_rbuf[
                b, pl.ds(j * CHUNK, CHUNK), :
            ]
        for c in range(n_chunks):
            y_rdmas[c].wait_send()
            x_rdmas[c].wait_send()

    return pl.pallas_call(
        body,
        out_shape=jax.ShapeDtypeStruct((B, S_half, N), jnp.float32),
        in_specs=[
            pl.BlockSpec(memory_space=pltpu.VMEM),
            pl.BlockSpec(memory_space=pltpu.VMEM),
        ],
        out_specs=pl.BlockSpec(memory_space=pltpu.VMEM),
        scratch_shapes=[
            pltpu.VMEM((B, Q, N), jnp.float32),
            pltpu.VMEM((B, Q, N), jnp.float32),
            pltpu.VMEM((B, Q, N), jnp.float32),
            pltpu.VMEM((B, Q, N), jnp.float32),
            pltpu.SemaphoreType.DMA((n_chunks,)),
            pltpu.SemaphoreType.DMA((n_chunks,)),
            pltpu.SemaphoreType.DMA((n_chunks,)),
            pltpu.SemaphoreType.DMA((n_chunks,)),
        ],
        compiler_params=pltpu.CompilerParams(collective_id=0),
    )(O, Wo.reshape(H, D, N))


# device time: 70603 ns/iter; 1.3335x vs baseline; 1.3335x over previous
import jax
import jax.numpy as jnp
from jax import lax
from jax.experimental import pallas as pl
from jax.experimental.pallas import tpu as pltpu


def kernel(O, Wo):
    B, S, H, D = O.shape
    HD = H * D
    N = Wo.shape[1]
    S_half = S // 2
    Q = S_half // 2

    O2 = O.reshape(B, S, HD)

    CHUNK = 32
    n_j = Q // CHUNK
    n_chunks = B * n_j

    def body(o_ref, wo_ref, out_ref, y_sbuf, y_rbuf, x_sbuf, x_rbuf,
             y_send_sems, y_recv_sems, x_send_sems, x_recv_sems):
        my_x = lax.axis_index("x")
        my_y = lax.axis_index("y")
        peer_y = 1 - my_y
        peer_x = 1 - my_x

        barrier = pltpu.get_barrier_semaphore()
        pl.semaphore_signal(
            barrier, inc=1,
            device_id=(my_x, peer_y), device_id_type=pl.DeviceIdType.MESH,
        )
        pl.semaphore_signal(
            barrier, inc=1,
            device_id=(peer_x, my_y), device_id_type=pl.DeviceIdType.MESH,
        )
        pl.semaphore_wait(barrier, 2)

        wo = wo_ref[:, :]

        y_rdmas = []
        for b in range(B):
            row0 = peer_y * S_half + my_x * Q
            y_sbuf[b, pl.ds(0, CHUNK), :] = jnp.dot(
                o_ref[b, pl.ds(row0, CHUNK), :], wo,
                preferred_element_type=jnp.float32,
            )
            y_rdmas.append(None)
            r0 = pltpu.make_async_remote_copy(
                src_ref=y_sbuf.at[b, pl.ds(0, CHUNK), :],
                dst_ref=y_rbuf.at[b, pl.ds(0, CHUNK), :],
                send_sem=y_send_sems.at[b * n_j],
                recv_sem=y_recv_sems.at[b * n_j],
                device_id=(my_x, peer_y),
                device_id_type=pl.DeviceIdType.MESH,
            )
            r0.start()
            y_rdmas[b * n_j] = r0
            y_sbuf[b, pl.ds(CHUNK, Q - CHUNK), :] = jnp.dot(
                o_ref[b, pl.ds(row0 + CHUNK, Q - CHUNK), :], wo,
                preferred_element_type=jnp.float32,
            )
            for j in range(1, n_j):
                r = pltpu.make_async_remote_copy(
                    src_ref=y_sbuf.at[b, pl.ds(j * CHUNK, CHUNK), :],
                    dst_ref=y_rbuf.at[b, pl.ds(j * CHUNK, CHUNK), :],
                    send_sem=y_send_sems.at[b * n_j + j],
                    recv_sem=y_recv_sems.at[b * n_j + j],
                    device_id=(my_x, peer_y),
                    device_id_type=pl.DeviceIdType.MESH,
                )
                r.start()
                y_rdmas.append(r)

        for b in range(B):
            row0 = my_y * S_half + my_x * Q
            out_ref[b, pl.ds(my_x * Q, Q), :] = jnp.dot(
                o_ref[b, pl.ds(row0, Q), :], wo,
                preferred_element_type=jnp.float32,
            )

        x_rdmas = []
        for c in range(n_chunks):
            b, j = divmod(c, n_j)
            out_row0 = my_x * Q + j * CHUNK
            y_rdmas[c].wait_recv()
            red = (
                out_ref[b, pl.ds(out_row0, CHUNK), :]
                + y_rbuf[b, pl.ds(j * CHUNK, CHUNK), :]
            )
            out_ref[b, pl.ds(out_row0, CHUNK), :] = red
            x_sbuf[b, pl.ds(j * CHUNK, CHUNK), :] = red
            r = pltpu.make_async_remote_copy(
                src_ref=x_sbuf.at[b, pl.ds(j * CHUNK, CHUNK), :],
                dst_ref=x_rbuf.at[b, pl.ds(j * CHUNK, CHUNK), :],
                send_sem=x_send_sems.at[c],
                recv_sem=x_recv_sems.at[c],
                device_id=(peer_x, my_y),
                device_id_type=pl.DeviceIdType.MESH,
            )
            r.start()
            x_rdmas.append(r)

        for c in range(n_chunks):
            b, j = divmod(c, n_j)
            x_rdmas[c].wait_recv()
            out_ref[b, pl.ds(peer_x * Q + j * CHUNK, CHUNK), :] = x_rbuf[
                b, pl.ds(j * CHUNK, CHUNK), :
            ]
        for c in range(n_chunks):
            y_rdmas[c].wait_send()
            x_rdmas[c].wait_send()

    return pl.pallas_call(
        body,
        out_shape=jax.ShapeDtypeStruct((B, S_half, N), jnp.float32),
        in_specs=[
            pl.BlockSpec(memory_space=pltpu.VMEM),
            pl.BlockSpec(memory_space=pltpu.VMEM),
        ],
        out_specs=pl.BlockSpec(memory_space=pltpu.VMEM),
        scratch_shapes=[
            pltpu.VMEM((B, Q, N), jnp.float32),
            pltpu.VMEM((B, Q, N), jnp.float32),
            pltpu.VMEM((B, Q, N), jnp.float32),
            pltpu.VMEM((B, Q, N), jnp.float32),
            pltpu.SemaphoreType.DMA((n_chunks,)),
            pltpu.SemaphoreType.DMA((n_chunks,)),
            pltpu.SemaphoreType.DMA((n_chunks,)),
            pltpu.SemaphoreType.DMA((n_chunks,)),
        ],
        compiler_params=pltpu.CompilerParams(collective_id=0),
    )(O2, Wo)


# device time: 70506 ns/iter; 1.3353x vs baseline; 1.0014x over previous
import jax
import jax.numpy as jnp
from jax import lax
from jax.experimental import pallas as pl
from jax.experimental.pallas import tpu as pltpu


def kernel(O, Wo):
    B, S, H, D = O.shape
    HD = H * D
    N = Wo.shape[1]
    S_half = S // 2
    Q = S_half // 2

    O2 = O.reshape(B, S, HD)

    CHUNK = 32
    n_j = Q // CHUNK
    n_chunks = B * n_j

    def body(o_ref, wo_ref, out_ref, y_sbuf, y_rbuf, x_sbuf, x_rbuf, own_buf,
             y_send_sems, y_recv_sems, x_send_sems, x_recv_sems, store_sems):
        my_x = lax.axis_index("x")
        my_y = lax.axis_index("y")
        peer_y = 1 - my_y
        peer_x = 1 - my_x

        barrier = pltpu.get_barrier_semaphore()
        pl.semaphore_signal(
            barrier, inc=1,
            device_id=(my_x, peer_y), device_id_type=pl.DeviceIdType.MESH,
        )
        pl.semaphore_signal(
            barrier, inc=1,
            device_id=(peer_x, my_y), device_id_type=pl.DeviceIdType.MESH,
        )
        pl.semaphore_wait(barrier, 2)

        wo = wo_ref[:, :]

        y_rdmas = []
        for b in range(B):
            row0 = peer_y * S_half + my_x * Q
            y_sbuf[b, pl.ds(0, CHUNK), :] = jnp.dot(
                o_ref[b, pl.ds(row0, CHUNK), :], wo,
                preferred_element_type=jnp.float32,
            )
            y_rdmas.append(None)
            r0 = pltpu.make_async_remote_copy(
                src_ref=y_sbuf.at[b, pl.ds(0, CHUNK), :],
                dst_ref=y_rbuf.at[b, pl.ds(0, CHUNK), :],
                send_sem=y_send_sems.at[b * n_j],
                recv_sem=y_recv_sems.at[b * n_j],
                device_id=(my_x, peer_y),
                device_id_type=pl.DeviceIdType.MESH,
            )
            r0.start()
            y_rdmas[b * n_j] = r0
            y_sbuf[b, pl.ds(CHUNK, Q - CHUNK), :] = jnp.dot(
                o_ref[b, pl.ds(row0 + CHUNK, Q - CHUNK), :], wo,
                preferred_element_type=jnp.float32,
            )
            for j in range(1, n_j):
                r = pltpu.make_async_remote_copy(
                    src_ref=y_sbuf.at[b, pl.ds(j * CHUNK, CHUNK), :],
                    dst_ref=y_rbuf.at[b, pl.ds(j * CHUNK, CHUNK), :],
                    send_sem=y_send_sems.at[b * n_j + j],
                    recv_sem=y_recv_sems.at[b * n_j + j],
                    device_id=(my_x, peer_y),
                    device_id_type=pl.DeviceIdType.MESH,
                )
                r.start()
                y_rdmas.append(r)

        for b in range(B):
            row0 = my_y * S_half + my_x * Q
            own_buf[b, :, :] = jnp.dot(
                o_ref[b, pl.ds(row0, Q), :], wo,
                preferred_element_type=jnp.float32,
            )

        pending = [None, None]

        def store_chunk(src, dst, k):
            if pending[k % 2] is not None:
                pending[k % 2].wait()
            cp = pltpu.make_async_copy(src, dst, store_sems.at[k % 2])
            cp.start()
            pending[k % 2] = cp

        x_rdmas = []
        for c in range(n_chunks):
            b, j = divmod(c, n_j)
            y_rdmas[c].wait_recv()
            x_sbuf[b, pl.ds(j * CHUNK, CHUNK), :] = (
                own_buf[b, pl.ds(j * CHUNK, CHUNK), :]
                + y_rbuf[b, pl.ds(j * CHUNK, CHUNK), :]
            )
            r = pltpu.make_async_remote_copy(
                src_ref=x_sbuf.at[b, pl.ds(j * CHUNK, CHUNK), :],
                dst_ref=x_rbuf.at[b, pl.ds(j * CHUNK, CHUNK), :],
                send_sem=x_send_sems.at[c],
                recv_sem=x_recv_sems.at[c],
                device_id=(peer_x, my_y),
                device_id_type=pl.DeviceIdType.MESH,
            )
            r.start()
            x_rdmas.append(r)
            store_chunk(
                x_sbuf.at[b, pl.ds(j * CHUNK, CHUNK), :],
                out_ref.at[b, pl.ds(my_x * Q + j * CHUNK, CHUNK), :],
                c,
            )

        for c in range(n_chunks):
            b, j = divmod(c, n_j)
            x_rdmas[c].wait_recv()
            store_chunk(
                x_rbuf.at[b, pl.ds(j * CHUNK, CHUNK), :],
                out_ref.at[b, pl.ds(peer_x * Q + j * CHUNK, CHUNK), :],
                n_chunks + c,
            )
        for p in pending:
            if p is not None:
                p.wait()
        for c in range(n_chunks):
            y_rdmas[c].wait_send()
            x_rdmas[c].wait_send()

    return pl.pallas_call(
        body,
        out_shape=jax.ShapeDtypeStruct((B, S_half, N), jnp.float32),
        in_specs=[
            pl.BlockSpec(memory_space=pltpu.VMEM),
            pl.BlockSpec(memory_space=pltpu.VMEM),
        ],
        out_specs=pl.BlockSpec(memory_space=pl.ANY),
        scratch_shapes=[
            pltpu.VMEM((B, Q, N), jnp.float32),
            pltpu.VMEM((B, Q, N), jnp.float32),
            pltpu.VMEM((B, Q, N), jnp.float32),
            pltpu.VMEM((B, Q, N), jnp.float32),
            pltpu.VMEM((B, Q, N), jnp.float32),
            pltpu.SemaphoreType.DMA((n_chunks,)),
            pltpu.SemaphoreType.DMA((n_chunks,)),
            pltpu.SemaphoreType.DMA((n_chunks,)),
            pltpu.SemaphoreType.DMA((n_chunks,)),
            pltpu.SemaphoreType.DMA((2,)),
        ],
        compiler_params=pltpu.CompilerParams(collective_id=0),
    )(O2, Wo)
